# baseline (speedup 1.0000x reference)
import functools

import jax
import jax.numpy as jnp
from jax import lax
from jax.experimental import pallas as pl
from jax.experimental.pallas import tpu as pltpu
from jax.experimental.pallas import tpu_sc as plsc

EMBED_DIM = 64
OUT_DIM = 2
NC = 2
NS = 16
NW = NC * NS
PROJ_BLK = 16384
G = 128


def _project_body(tt_ref, w_ref, b_ref, pk_ref):
    x = tt_ref[...].astype(jnp.bfloat16)
    z = lax.dot_general(w_ref[...], x, (((1,), (0,)), ((), ())),
                        preferred_element_type=jnp.float32)
    t = jnp.tanh(z + b_ref[...]).astype(jnp.bfloat16)
    u = lax.bitcast_convert_type(t, jnp.uint16).astype(jnp.uint32)
    pk_ref[...] = u[0] | (u[1] << 16)


@functools.cache
def _make_project(n_rows: int):
    grid = (n_rows + PROJ_BLK - 1) // PROJ_BLK
    return pl.pallas_call(
        _project_body,
        grid=(grid,),
        in_specs=[
            pl.BlockSpec((EMBED_DIM, PROJ_BLK), lambda i: (0, i)),
            pl.BlockSpec((OUT_DIM, EMBED_DIM), lambda i: (0, 0)),
            pl.BlockSpec((OUT_DIM, 1), lambda i: (0, 0)),
        ],
        out_specs=pl.BlockSpec((PROJ_BLK,), lambda i: (i,)),
        out_shape=jax.ShapeDtypeStruct((n_rows,), jnp.uint32),
    )


@functools.cache
def _make_gather(fields: int, batch: int):
    bw = batch // NW
    qn = bw // G
    ch = fields * qn

    mesh = plsc.VectorSubcoreMesh(core_axis_name="c", subcore_axis_name="s")

    @functools.partial(
        pl.kernel,
        out_type=jax.ShapeDtypeStruct((fields, OUT_DIM, batch), jnp.float32),
        mesh=mesh,
        scratch_types=[
            pltpu.VMEM((fields, bw), jnp.int32),
            pltpu.VMEM((fields, bw), jnp.uint32),
            pltpu.VMEM((fields, OUT_DIM, bw), jnp.float32),
            pltpu.SemaphoreType.DMA,
        ],
        compiler_params=pltpu.CompilerParams(
            use_tc_tiling_on_sc=False, needs_layout_passes=False),
    )
    def gather2(idx_hbm, pk_hbm, out_hbm, idx_v, pk_v, out_v, sem):
        wid = lax.axis_index("s") * NC + lax.axis_index("c")
        base = wid * bw
        pltpu.sync_copy(idx_hbm.at[:, pl.ds(base, bw)], idx_v)

        def fire(j, _):
            f = j // qn
            q = j % qn
            pltpu.async_copy(pk_hbm.at[idx_v.at[f, pl.ds(q * G, G)]],
                             pk_v.at[f, pl.ds(q * G, G)], sem)
            return 0

        lax.fori_loop(0, ch, fire, 0)

        def drain(j, _):
            pltpu.make_async_copy(
                pk_hbm.at[idx_v.at[0, pl.ds(0, G)]],
                pk_v.at[0, pl.ds(0, G)], sem).wait()
            return 0

        lax.fori_loop(0, ch, drain, 0)

        hi_mask = jnp.full((16,), 0xFFFF0000, jnp.uint32)

        def unpack(i, _):
            f = i // (bw // 16)
            s = (i % (bw // 16)) * 16
            v = pk_v[f, pl.ds(s, 16)]
            out_v[f, 0, pl.ds(s, 16)] = plsc.bitcast(v << 16, jnp.float32)
            out_v[f, 1, pl.ds(s, 16)] = plsc.bitcast(v & hi_mask, jnp.float32)
            return 0

        lax.fori_loop(0, fields * (bw // 16), unpack, 0)

        pltpu.sync_copy(out_v, out_hbm.at[:, :, pl.ds(base, bw)])

    return gather2


def kernel(data, table, W, b):
    batch, fields = data.shape
    table_t = table.T
    data_t = data.astype(jnp.int32).T
    pk = _make_project(table.shape[0])(
        table_t, W.astype(jnp.bfloat16), b.reshape(OUT_DIM, 1))
    out3 = _make_gather(fields, batch)(data_t, pk)
    return out3.transpose(2, 0, 1)


# speedup vs baseline: 1.0811x; 1.0811x over previous
import functools

import jax
import jax.numpy as jnp
from jax import lax
from jax.experimental import pallas as pl
from jax.experimental.pallas import tpu as pltpu
from jax.experimental.pallas import tpu_sc as plsc

EMBED_DIM = 64
OUT_DIM = 2
NC = 2
NS = 16
NW = NC * NS
PROJ_BLK = 40960
G = 128


def _project_body(tt_ref, w_ref, b_ref, pk_ref):
    x = tt_ref[...].astype(jnp.bfloat16)
    z = lax.dot_general(w_ref[...], x, (((1,), (0,)), ((), ())),
                        preferred_element_type=jnp.float32)
    t = jnp.tanh(z + b_ref[...]).astype(jnp.bfloat16)
    u = lax.bitcast_convert_type(t, jnp.uint16).astype(jnp.uint32)
    pk_ref[...] = u[0] | (u[1] << 16)


@functools.cache
def _make_project(n_rows: int):
    grid = (n_rows + PROJ_BLK - 1) // PROJ_BLK
    return pl.pallas_call(
        _project_body,
        grid=(grid,),
        in_specs=[
            pl.BlockSpec((EMBED_DIM, PROJ_BLK), lambda i: (0, i)),
            pl.BlockSpec((OUT_DIM, EMBED_DIM), lambda i: (0, 0)),
            pl.BlockSpec((OUT_DIM, 1), lambda i: (0, 0)),
        ],
        out_specs=pl.BlockSpec((PROJ_BLK,), lambda i: (i,)),
        out_shape=jax.ShapeDtypeStruct((n_rows,), jnp.uint32),
    )


@functools.cache
def _make_gather(fields: int, batch: int):
    bw = batch // NW
    qn = bw // G
    ch = fields * qn

    mesh = plsc.VectorSubcoreMesh(core_axis_name="c", subcore_axis_name="s")

    @functools.partial(
        pl.kernel,
        out_type=jax.ShapeDtypeStruct((fields, OUT_DIM, batch), jnp.float32),
        mesh=mesh,
        scratch_types=[
            pltpu.VMEM((fields, bw), jnp.int32),
            pltpu.VMEM((fields, bw), jnp.uint32),
            pltpu.VMEM((fields, OUT_DIM, bw), jnp.float32),
            pltpu.SemaphoreType.DMA,
        ],
        compiler_params=pltpu.CompilerParams(
            use_tc_tiling_on_sc=False, needs_layout_passes=False),
    )
    def gather2(idx_hbm, pk_hbm, out_hbm, idx_v, pk_v, out_v, sem):
        wid = lax.axis_index("s") * NC + lax.axis_index("c")
        base = wid * bw
        pltpu.sync_copy(idx_hbm.at[:, pl.ds(base, bw)], idx_v)

        def fire(j, _):
            f = j // qn
            q = j % qn
            pltpu.async_copy(pk_hbm.at[idx_v.at[f, pl.ds(q * G, G)]],
                             pk_v.at[f, pl.ds(q * G, G)], sem)
            return 0

        lax.fori_loop(0, ch, fire, 0)

        def drain(j, _):
            pltpu.make_async_copy(
                pk_hbm.at[idx_v.at[0, pl.ds(0, G)]],
                pk_v.at[0, pl.ds(0, G)], sem).wait()
            return 0

        lax.fori_loop(0, ch, drain, 0)

        hi_mask = jnp.full((16,), 0xFFFF0000, jnp.uint32)

        def unpack(i, _):
            f = i // (bw // 16)
            s = (i % (bw // 16)) * 16
            v = pk_v[f, pl.ds(s, 16)]
            out_v[f, 0, pl.ds(s, 16)] = plsc.bitcast(v << 16, jnp.float32)
            out_v[f, 1, pl.ds(s, 16)] = plsc.bitcast(v & hi_mask, jnp.float32)
            return 0

        lax.fori_loop(0, fields * (bw // 16), unpack, 0)

        pltpu.sync_copy(out_v, out_hbm.at[:, :, pl.ds(base, bw)])

    return gather2


def kernel(data, table, W, b):
    batch, fields = data.shape
    table_t = table.T
    data_t = data.astype(jnp.int32).T
    pk = _make_project(table.shape[0])(
        table_t, W.astype(jnp.bfloat16), b.reshape(OUT_DIM, 1))
    out3 = _make_gather(fields, batch)(data_t, pk)
    return out3.transpose(2, 0, 1)


# speedup vs baseline: 1.0860x; 1.0046x over previous
import functools

import jax
import jax.numpy as jnp
from jax import lax
from jax.experimental import pallas as pl
from jax.experimental.pallas import tpu as pltpu
from jax.experimental.pallas import tpu_sc as plsc

EMBED_DIM = 64
OUT_DIM = 2
NC = 2
NS = 16
NW = NC * NS
PROJ_BLK = 32768
G = 128


def _project_body(tt_ref, w_ref, b_ref, pk_ref):
    x = tt_ref[...].astype(jnp.bfloat16)
    z = lax.dot_general(w_ref[...], x, (((1,), (0,)), ((), ())),
                        preferred_element_type=jnp.float32)
    t = jnp.tanh(z + b_ref[...]).astype(jnp.bfloat16)
    u = lax.bitcast_convert_type(t, jnp.uint16).astype(jnp.uint32)
    pk_ref[...] = u[0] | (u[1] << 16)


@functools.cache
def _make_project(n_rows: int):
    grid = (n_rows + PROJ_BLK - 1) // PROJ_BLK
    return pl.pallas_call(
        _project_body,
        grid=(grid,),
        in_specs=[
            pl.BlockSpec((EMBED_DIM, PROJ_BLK), lambda i: (0, i)),
            pl.BlockSpec((OUT_DIM, EMBED_DIM), lambda i: (0, 0)),
            pl.BlockSpec((OUT_DIM, 1), lambda i: (0, 0)),
        ],
        out_specs=pl.BlockSpec((PROJ_BLK,), lambda i: (i,)),
        out_shape=jax.ShapeDtypeStruct((n_rows,), jnp.uint32),
    )


@functools.cache
def _make_gather(fields: int, batch: int):
    bw = batch // NW
    qn = bw // G
    ch = fields * qn

    mesh = plsc.VectorSubcoreMesh(core_axis_name="c", subcore_axis_name="s")

    @functools.partial(
        pl.kernel,
        out_type=jax.ShapeDtypeStruct((fields, OUT_DIM, batch), jnp.float32),
        mesh=mesh,
        scratch_types=[
            pltpu.VMEM((fields, bw), jnp.int32),
            pltpu.VMEM((fields, bw), jnp.uint32),
            pltpu.VMEM((fields, OUT_DIM, bw), jnp.float32),
            pltpu.SemaphoreType.DMA,
        ],
        compiler_params=pltpu.CompilerParams(
            use_tc_tiling_on_sc=False, needs_layout_passes=False),
    )
    def gather2(idx_hbm, pk_hbm, out_hbm, idx_v, pk_v, out_v, sem):
        wid = lax.axis_index("s") * NC + lax.axis_index("c")
        base = wid * bw
        pltpu.sync_copy(idx_hbm.at[:, pl.ds(base, bw)], idx_v)

        def fire(j, _):
            f = j // qn
            q = j % qn
            pltpu.async_copy(pk_hbm.at[idx_v.at[f, pl.ds(q * G, G)]],
                             pk_v.at[f, pl.ds(q * G, G)], sem)
            return 0

        lax.fori_loop(0, ch, fire, 0)

        def drain(j, _):
            pltpu.make_async_copy(
                pk_hbm.at[idx_v.at[0, pl.ds(0, G)]],
                pk_v.at[0, pl.ds(0, G)], sem).wait()
            return 0

        lax.fori_loop(0, ch, drain, 0)

        hi_mask = jnp.full((16,), 0xFFFF0000, jnp.uint32)

        def unpack(i, _):
            f = i // (bw // 16)
            s = (i % (bw // 16)) * 16
            v = pk_v[f, pl.ds(s, 16)]
            out_v[f, 0, pl.ds(s, 16)] = plsc.bitcast(v << 16, jnp.float32)
            out_v[f, 1, pl.ds(s, 16)] = plsc.bitcast(v & hi_mask, jnp.float32)
            return 0

        lax.fori_loop(0, fields * (bw // 16), unpack, 0)

        pltpu.sync_copy(out_v, out_hbm.at[:, :, pl.ds(base, bw)])

    return gather2


def kernel(data, table, W, b):
    batch, fields = data.shape
    table_t = table.T
    data_t = data.astype(jnp.int32).T
    pk = _make_project(table.shape[0])(
        table_t, W.astype(jnp.bfloat16), b.reshape(OUT_DIM, 1))
    out3 = _make_gather(fields, batch)(data_t, pk)
    return out3.transpose(2, 0, 1)
